# baseline (speedup 1.0000x reference)
import jax
import jax.numpy as jnp
from jax import lax
from jax.experimental import pallas as pl
from jax.experimental.pallas import tpu as pltpu


def _w2_body(w_ref, w2_ref):
    wv = w_ref[...]
    w2_ref[...] = jnp.sum(wv * wv, axis=1)[None, :]


def _vq_body(x_ref, w_ref, w2_ref, dist_ref, p_ref, q_ref, xout_ref):
    x = x_ref[...]
    w = w_ref[...]
    w2 = w2_ref[...]
    x2 = jnp.sum(x * x, axis=1, keepdims=True)
    xw2 = lax.dot_general(x + x, w, (((1,), (1,)), ((), ())),
                          preferred_element_type=jnp.float32)
    u = xw2 - w2
    dist_ref[...] = x2 - u
    m = jnp.max(u, axis=1, keepdims=True)
    e = jnp.exp(u - m)
    s = jnp.sum(e, axis=1, keepdims=True)
    p = e * (1.0 / s)
    p_ref[...] = p
    q_ref[...] = lax.dot_general(p, w, (((1,), (0,)), ((), ())),
                                 preferred_element_type=jnp.float32)
    xout_ref[...] = x


def kernel(feat, vq_weight):
    b, c, h, w = feat.shape
    k = vq_weight.shape[0]
    n = b * h * w
    r_tile = 192
    nr = n // r_tile
    flat = jnp.transpose(feat, (0, 2, 3, 1)).reshape(n, c)

    w2 = pl.pallas_call(
        _w2_body,
        out_shape=jax.ShapeDtypeStruct((1, k), jnp.float32),
    )(vq_weight)

    dist, p_flat, q, x_out = pl.pallas_call(
        _vq_body,
        grid=(nr,),
        in_specs=[
            pl.BlockSpec((r_tile, c), lambda i: (i, 0)),
            pl.BlockSpec((k, c), lambda i: (0, 0)),
            pl.BlockSpec((1, k), lambda i: (0, 0)),
        ],
        out_specs=[
            pl.BlockSpec((r_tile, k), lambda i: (i, 0)),
            pl.BlockSpec((r_tile, k), lambda i: (i, 0)),
            pl.BlockSpec((r_tile, c), lambda i: (i, 0)),
            pl.BlockSpec((r_tile, c), lambda i: (i, 0)),
        ],
        out_shape=[
            jax.ShapeDtypeStruct((n, k), jnp.float32),
            jax.ShapeDtypeStruct((n, k), jnp.float32),
            jax.ShapeDtypeStruct((n, c), jnp.float32),
            jax.ShapeDtypeStruct((n, c), jnp.float32),
        ],
        compiler_params=pltpu.CompilerParams(
            dimension_semantics=("parallel",),
        ),
    )(flat, vq_weight, w2)

    featp = x_out.reshape(b, h, w, c)
    q_feat = jnp.transpose(q.reshape(b, h, w, c), (0, 3, 1, 2))
    assignment = jnp.transpose(p_flat.reshape(b, h, w, k), (0, 3, 1, 2))
    return (featp, q_feat, assignment, dist)


# speedup vs baseline: 1.0989x; 1.0989x over previous
import jax
import jax.numpy as jnp
from jax import lax
from jax.experimental import pallas as pl
from jax.experimental.pallas import tpu as pltpu


def _vq_body(x_ref, w_ref, dist_ref, p_ref, q_ref, xout_ref):
    x = x_ref[...]
    w = w_ref[...]
    x2 = jnp.sum(x * x, axis=1, keepdims=True)
    w2 = jnp.sum(w * w, axis=1)
    xw = lax.dot_general(x, w, (((1,), (1,)), ((), ())),
                         preferred_element_type=jnp.float32)
    dist = x2 + w2[None, :] - 2.0 * xw
    dist_ref[...] = dist
    neg = -dist
    m = jnp.max(neg, axis=1, keepdims=True)
    e = jnp.exp(neg - m)
    s_inv = 1.0 / jnp.sum(e, axis=1, keepdims=True)
    p_ref[...] = e * s_inv
    ew = lax.dot_general(e, w, (((1,), (0,)), ((), ())),
                         preferred_element_type=jnp.float32)
    q_ref[...] = ew * s_inv
    xout_ref[...] = x


def kernel(feat, vq_weight):
    b, c, h, w = feat.shape
    k = vq_weight.shape[0]
    n = b * h * w
    r_tile = 192
    nr = n // r_tile
    flat = jnp.transpose(feat, (0, 2, 3, 1)).reshape(n, c)

    dist, p_flat, q, x_out = pl.pallas_call(
        _vq_body,
        grid=(nr,),
        in_specs=[
            pl.BlockSpec((r_tile, c), lambda i: (i, 0)),
            pl.BlockSpec((k, c), lambda i: (0, 0)),
        ],
        out_specs=[
            pl.BlockSpec((r_tile, k), lambda i: (i, 0)),
            pl.BlockSpec((r_tile, k), lambda i: (i, 0)),
            pl.BlockSpec((r_tile, c), lambda i: (i, 0)),
            pl.BlockSpec((r_tile, c), lambda i: (i, 0)),
        ],
        out_shape=[
            jax.ShapeDtypeStruct((n, k), jnp.float32),
            jax.ShapeDtypeStruct((n, k), jnp.float32),
            jax.ShapeDtypeStruct((n, c), jnp.float32),
            jax.ShapeDtypeStruct((n, c), jnp.float32),
        ],
        compiler_params=pltpu.CompilerParams(
            dimension_semantics=("parallel",),
        ),
    )(flat, vq_weight)

    featp = x_out.reshape(b, h, w, c)
    q_feat = jnp.transpose(q.reshape(b, h, w, c), (0, 3, 1, 2))
    assignment = jnp.transpose(p_flat.reshape(b, h, w, k), (0, 3, 1, 2))
    return (featp, q_feat, assignment, dist)


# speedup vs baseline: 1.1497x; 1.0462x over previous
import jax
import jax.numpy as jnp
from jax import lax
from jax.experimental import pallas as pl
from jax.experimental.pallas import tpu as pltpu


def _vq_body(x_ref, w_ref, dist_ref, p_ref, q_ref, xout_ref):
    x = x_ref[...]
    w = w_ref[...]
    x2 = jnp.sum(x * x, axis=1, keepdims=True)
    w2 = jnp.sum(w * w, axis=1)
    xw = lax.dot_general(x, w, (((1,), (1,)), ((), ())),
                         preferred_element_type=jnp.float32)
    dist = x2 + w2[None, :] - 2.0 * xw
    dist_ref[...] = dist
    neg = -dist
    m = jnp.max(neg, axis=1, keepdims=True)
    e = jnp.exp(neg - m)
    s_inv = 1.0 / jnp.sum(e, axis=1, keepdims=True)
    p_ref[...] = e * s_inv
    ew = lax.dot_general(e, w, (((1,), (0,)), ((), ())),
                         preferred_element_type=jnp.float32)
    q_ref[...] = ew * s_inv
    xout_ref[...] = x


def kernel(feat, vq_weight):
    b, c, h, w = feat.shape
    k = vq_weight.shape[0]
    n = b * h * w
    r_tile = 256
    nr = n // r_tile
    flat = jnp.transpose(feat, (0, 2, 3, 1)).reshape(n, c)

    dist, p_flat, q, x_out = pl.pallas_call(
        _vq_body,
        grid=(nr,),
        in_specs=[
            pl.BlockSpec((r_tile, c), lambda i: (i, 0)),
            pl.BlockSpec((k, c), lambda i: (0, 0)),
        ],
        out_specs=[
            pl.BlockSpec((r_tile, k), lambda i: (i, 0)),
            pl.BlockSpec((r_tile, k), lambda i: (i, 0)),
            pl.BlockSpec((r_tile, c), lambda i: (i, 0)),
            pl.BlockSpec((r_tile, c), lambda i: (i, 0)),
        ],
        out_shape=[
            jax.ShapeDtypeStruct((n, k), jnp.float32),
            jax.ShapeDtypeStruct((n, k), jnp.float32),
            jax.ShapeDtypeStruct((n, c), jnp.float32),
            jax.ShapeDtypeStruct((n, c), jnp.float32),
        ],
        compiler_params=pltpu.CompilerParams(
            dimension_semantics=("parallel",),
        ),
    )(flat, vq_weight)

    featp = x_out.reshape(b, h, w, c)
    q_feat = jnp.transpose(q.reshape(b, h, w, c), (0, 3, 1, 2))
    assignment = jnp.transpose(p_flat.reshape(b, h, w, k), (0, 3, 1, 2))
    return (featp, q_feat, assignment, dist)
